# baseline (speedup 1.0000x reference)
import jax
import jax.numpy as jnp
from jax import lax
from jax.experimental import pallas as pl
from jax.experimental.pallas import tpu as pltpu
from jax.experimental.pallas import tpu_sc as plsc

N = 10000
D = 128
DH = D // 2
NC = 2
NS = 16
NTILE = NC * NS
NPAD = 10240
RPT = NPAD // NS
SPT = N // NS
C = 64
EPT = 10240
NCHUNK = EPT // C
EPAD = NTILE * EPT
NB = 6
LOOKA = NB // 2
F32 = jnp.float32
I32 = jnp.int32

_SC_PARAMS = pltpu.CompilerParams(needs_layout_passes=False,
                                  use_tc_tiling_on_sc=False)


def _rsqrt16(d):
    i = lax.bitcast_convert_type(d, I32)
    i = jnp.int32(0x5F3759DF) - (i >> 1)
    y = lax.bitcast_convert_type(i, F32)
    for _ in range(3):
        y = y * (1.5 - 0.5 * d * y * y)
    return y


def _sc_deg_body(row_hbm, col_hbm, x_hbm, colp_hbm, dis_hbm, slo_hbm, shi_hbm,
                 row_v, col_v, hist, shared, slab, dis_v, xbuf, xlo, xhi):
    c = lax.axis_index("c")
    s = lax.axis_index("s")

    zero16 = jnp.zeros((16,), F32)
    ones16 = jnp.ones((16,), F32)
    n16 = jnp.full((16,), N, I32)
    dummy16 = N + lax.iota(I32, 16)

    def zinit(i, carry):
        hist[pl.ds(i * 16, 16)] = zero16
        return carry
    lax.fori_loop(0, NPAD // 16, zinit, 0)

    for h in range(NC):
        pltpu.sync_copy(row_hbm.at[h, s], row_v)
        pltpu.sync_copy(col_hbm.at[h, s], col_v)

        def hbody(j, carry):
            for k in range(C // 16):
                r = row_v[pl.ds(j * C + k * 16, 16)]
                cc = col_v[j, pl.ds(k * 16, 16)]
                m = (r != cc) & (cc < n16)
                plsc.addupdate_scatter(hist, [r], ones16, mask=m)
            return carry
        lax.fori_loop(0, NCHUNK, hbody, 0)

    pltpu.sync_copy(row_hbm.at[c, s], row_v)
    pltpu.sync_copy(col_hbm.at[c, s], col_v)

    def ebody(j, carry):
        for k in range(C // 16):
            r = row_v[pl.ds(j * C + k * 16, 16)]
            cc = col_v[j, pl.ds(k * 16, 16)]
            col_v[j, pl.ds(k * 16, 16)] = jnp.where(r != cc, cc, dummy16)
        return carry
    lax.fori_loop(0, NCHUNK, ebody, 0)
    pltpu.sync_copy(col_v, colp_hbm.at[c, s])

    pltpu.sync_copy(hist, shared.at[s])
    plsc.subcore_barrier()
    for t in range(NS):
        pltpu.sync_copy(shared.at[t, pl.ds(s * RPT, RPT)], slab.at[t])

    def rbody(i, carry):
        a = slab[0, pl.ds(i * 16, 16)]
        for t in range(1, NS):
            a = a + slab[t, pl.ds(i * 16, 16)]
        dis_v[pl.ds(i * 16, 16)] = jnp.where(a > 0.0, _rsqrt16(a), zero16)
        return carry
    lax.fori_loop(0, RPT // 16, rbody, 0)
    pltpu.sync_copy(dis_v.at[pl.ds(0, RPT)], dis_hbm.at[c, pl.ds(s * RPT, RPT)])

    for q in range(RPT // (2 * C)):
        r0 = s * RPT + 320 * c + q * C
        l0 = 320 * c + q * C
        pltpu.sync_copy(x_hbm.at[pl.ds(r0, C)], xbuf)

        def sbody(i, carry):
            dv = dis_v[pl.ds(l0 + i, 16)][0]
            for k in range(D // 16):
                v = xbuf[i, pl.ds(k * 16, 16)] * dv
                if k < DH // 16:
                    xlo[i, pl.ds(k * 16, 16)] = v
                else:
                    xhi[i, pl.ds((k - DH // 16) * 16, 16)] = v
            return carry
        lax.fori_loop(0, C, sbody, 0)
        pltpu.sync_copy(xlo, slo_hbm.at[pl.ds(r0, C)])
        pltpu.sync_copy(xhi, shi_hbm.at[pl.ds(r0, C)])


def _make_sc_deg(mesh):
    return pl.kernel(
        _sc_deg_body,
        out_type=(jax.ShapeDtypeStruct((NC, NS, NCHUNK, C), I32),
                  jax.ShapeDtypeStruct((NC, NPAD), F32),
                  jax.ShapeDtypeStruct((NPAD, DH), F32),
                  jax.ShapeDtypeStruct((NPAD, DH), F32)),
        mesh=mesh,
        compiler_params=_SC_PARAMS,
        scratch_types=[
            pltpu.VMEM((EPT,), I32),
            pltpu.VMEM((NCHUNK, C), I32),
            pltpu.VMEM((NPAD,), F32),
            pltpu.VMEM_SHARED((NS, NPAD), F32),
            pltpu.VMEM((NS, RPT), F32),
            pltpu.VMEM((RPT + 16,), F32),
            pltpu.VMEM((C, D), F32),
            pltpu.VMEM((C, DH), F32),
            pltpu.VMEM((C, DH), F32),
        ],
    )


def _prop_mainloop(s, table, acc, row_v, colp_v, bufs, gsems, ssems):
    for i in range(RPT // C):
        pltpu.sync_copy(bufs[0], acc.at[pl.ds(s * RPT + i * C, C)])
    plsc.subcore_barrier()

    nround = (NCHUNK + LOOKA + NB) // NB + 1

    def round_(g, carry):
        for b in range(NB):
            k = g * NB + b

            @pl.when((k >= NB) & (k < NCHUNK + NB))
            def _():
                pltpu.make_async_copy(
                    bufs[b], acc.at[colp_v.at[k - NB]], ssems[b]).wait()

            @pl.when(k < NCHUNK)
            def _():
                pltpu.async_copy(
                    table.at[row_v.at[pl.ds(k * C, C)]], bufs[b], gsems[b])

            j = k - LOOKA
            bj = (b - LOOKA) % NB

            @pl.when((j >= 0) & (j < NCHUNK))
            def _():
                pltpu.make_async_copy(
                    table.at[row_v.at[pl.ds(j * C, C)]], bufs[bj],
                    gsems[bj]).wait()
                pltpu.async_copy(bufs[bj], acc.at[colp_v.at[j]],
                                 ssems[bj], add=True)
        return carry
    lax.fori_loop(0, nround, round_, 0)
    plsc.subcore_barrier()


def _zero_seed(buf):
    zero16 = jnp.zeros((16,), F32)

    def zb(i, carry):
        for k in range(DH // 16):
            buf[i, pl.ds(k * 16, 16)] = zero16
        return carry
    lax.fori_loop(0, C, zb, 0)


def _sc_prop1_body(slo_hbm, shi_hbm, row_hbm, colp_hbm, r_hbm,
                   row_v, colp_v, *rest):
    bufs = rest[:NB]
    table, acc = rest[NB], rest[NB + 1]
    gsems = rest[NB + 2:NB + 2 + NB]
    ssems = rest[NB + 2 + NB:]
    c = lax.axis_index("c")
    s = lax.axis_index("s")
    pltpu.sync_copy(row_hbm.at[c, s], row_v)
    pltpu.sync_copy(colp_hbm.at[c, s], colp_v)
    _zero_seed(bufs[0])

    for p, s_hbm in enumerate((slo_hbm, shi_hbm)):
        pltpu.sync_copy(s_hbm.at[pl.ds(s * SPT, SPT)],
                        table.at[pl.ds(s * SPT, SPT)])
        _prop_mainloop(s, table, acc, row_v, colp_v, bufs, gsems, ssems)
        pltpu.sync_copy(acc.at[pl.ds(s * RPT, RPT)],
                        r_hbm.at[c, p, pl.ds(s * RPT, RPT)])
        if p == 0:
            _zero_seed(bufs[0])
            plsc.subcore_barrier()


def _make_sc_prop1(mesh):
    return pl.kernel(
        _sc_prop1_body,
        out_type=jax.ShapeDtypeStruct((NC, 2, NPAD, DH), F32),
        mesh=mesh,
        compiler_params=_SC_PARAMS,
        scratch_types=(
            [pltpu.VMEM((EPT,), I32),
             pltpu.VMEM((NCHUNK, C), I32)]
            + [pltpu.VMEM((C, DH), F32) for _ in range(NB)]
            + [pltpu.VMEM_SHARED((N, DH), F32),
               pltpu.VMEM_SHARED((NPAD, DH), F32)]
            + [pltpu.SemaphoreType.DMA for _ in range(2 * NB)]
        ),
    )


DISW = 656


def _sc_prop2_body(r1_hbm, dis_hbm, row_hbm, colp_hbm, r_hbm,
                   row_v, colp_v, dis_w, *rest):
    bufs = rest[:NB]
    table, acc = rest[NB], rest[NB + 1]
    gsems = rest[NB + 2:NB + 2 + NB]
    ssems = rest[NB + 2 + NB:]
    c = lax.axis_index("c")
    s = lax.axis_index("s")
    pltpu.sync_copy(row_hbm.at[c, s], row_v)
    pltpu.sync_copy(colp_hbm.at[c, s], colp_v)
    pltpu.sync_copy(dis_hbm.at[c, pl.ds(s * 624, DISW)], dis_w)

    for p in range(2):
        nfull = SPT // C
        for q in range(nfull + 1):
            cl = C if q < nfull else SPT - nfull * C
            r0 = s * SPT + q * C
            pltpu.sync_copy(r1_hbm.at[0, p, pl.ds(r0, cl)], bufs[0].at[pl.ds(0, cl)])
            pltpu.sync_copy(r1_hbm.at[1, p, pl.ds(r0, cl)], bufs[1].at[pl.ds(0, cl)])

            def tbody(i, carry):
                dv = dis_w[pl.ds(s + q * C + i, 16)][0]
                f = -(dv * dv)
                for k in range(DH // 16):
                    bufs[0][i, pl.ds(k * 16, 16)] = (
                        bufs[0][i, pl.ds(k * 16, 16)]
                        + bufs[1][i, pl.ds(k * 16, 16)]) * f
                return carry
            lax.fori_loop(0, cl, tbody, 0)
            pltpu.sync_copy(bufs[0].at[pl.ds(0, cl)], table.at[pl.ds(r0, cl)])

        _zero_seed(bufs[0])
        _prop_mainloop(s, table, acc, row_v, colp_v, bufs, gsems, ssems)
        pltpu.sync_copy(acc.at[pl.ds(s * RPT, RPT)],
                        r_hbm.at[c, p, pl.ds(s * RPT, RPT)])
        if p == 0:
            plsc.subcore_barrier()


def _make_sc_prop2(mesh):
    return pl.kernel(
        _sc_prop2_body,
        out_type=jax.ShapeDtypeStruct((NC, 2, NPAD, DH), F32),
        mesh=mesh,
        compiler_params=_SC_PARAMS,
        scratch_types=(
            [pltpu.VMEM((EPT,), I32),
             pltpu.VMEM((NCHUNK, C), I32),
             pltpu.VMEM((DISW,), F32)]
            + [pltpu.VMEM((C, DH), F32) for _ in range(NB)]
            + [pltpu.VMEM_SHARED((N, DH), F32),
               pltpu.VMEM_SHARED((NPAD, DH), F32)]
            + [pltpu.SemaphoreType.DMA for _ in range(2 * NB)]
        ),
    )


BR = 2000


def _tc_c_body(x_ref, r1_ref, r2_ref, dis_ref, w_ref, b_ref, out_ref):
    x = x_ref[...]
    dis = dis_ref[...]
    tx1 = jnp.concatenate(
        [(r1_ref[0, 0] + r1_ref[1, 0]),
         (r1_ref[0, 1] + r1_ref[1, 1])], axis=1) * (-dis)
    tx2 = jnp.concatenate(
        [(r2_ref[0, 0] + r2_ref[1, 0]),
         (r2_ref[0, 1] + r2_ref[1, 1])], axis=1) * (-2.0 * dis) - x
    out = jnp.dot(x, w_ref[0], preferred_element_type=F32)
    out = out + jnp.dot(tx1, w_ref[1], preferred_element_type=F32)
    out = out + jnp.dot(tx2, w_ref[2], preferred_element_type=F32)
    out_ref[...] = out + b_ref[...]


_tc_c = pl.pallas_call(
    _tc_c_body,
    grid=(N // BR,),
    in_specs=[
        pl.BlockSpec((BR, D), lambda i: (i, 0)),
        pl.BlockSpec((NC, 2, BR, DH), lambda i: (0, 0, i, 0)),
        pl.BlockSpec((NC, 2, BR, DH), lambda i: (0, 0, i, 0)),
        pl.BlockSpec((BR, 1), lambda i: (i, 0)),
        pl.BlockSpec((3, D, D), lambda i: (0, 0, 0)),
        pl.BlockSpec((1, D), lambda i: (0, 0)),
    ],
    out_specs=pl.BlockSpec((BR, D), lambda i: (i, 0)),
    out_shape=jax.ShapeDtypeStruct((N, D), F32),
)


def kernel(x, edge_index, W, b):
    row = edge_index[0].astype(I32)
    col = edge_index[1].astype(I32)
    e = row.shape[0]
    ept_real = e // NTILE
    ppt = EPT - ept_real
    pad_col = (N + jnp.arange(NTILE * ppt, dtype=I32) % (NPAD - N)).reshape(NTILE, ppt)
    row_t = jnp.concatenate(
        [row.reshape(NTILE, ept_real), jnp.zeros((NTILE, ppt), I32)],
        axis=1).reshape(NC, NS, EPT)
    col_t = jnp.concatenate(
        [col.reshape(NTILE, ept_real), pad_col],
        axis=1).reshape(NC, NS, NCHUNK, C)
    x_pad = jnp.concatenate([x, jnp.zeros((NPAD - N, D), F32)])

    mesh = plsc.VectorSubcoreMesh(core_axis_name="c", subcore_axis_name="s")
    colp_t, dis2, slo, shi = _make_sc_deg(mesh)(row_t, col_t, x_pad)
    r1 = _make_sc_prop1(mesh)(slo, shi, row_t, colp_t)
    r2 = _make_sc_prop2(mesh)(r1, dis2, row_t, colp_t)
    out = _tc_c(x, r1, r2, dis2[0, :N].reshape(N, 1), W, b.reshape(1, D))
    return out


# speedup vs baseline: 1.0964x; 1.0964x over previous
import jax
import jax.numpy as jnp
from jax import lax
from jax.experimental import pallas as pl
from jax.experimental.pallas import tpu as pltpu
from jax.experimental.pallas import tpu_sc as plsc

N = 10000
D = 128
DH = D // 2
NC = 2
NS = 16
NTILE = NC * NS
NPAD = 10240
RPT = NPAD // NS
SPT = N // NS
C = 64
EPT = 10240
NCHUNK = EPT // C
EPAD = NTILE * EPT
NB = 6
LOOKA = NB // 2
F32 = jnp.float32
I32 = jnp.int32

_SC_PARAMS = pltpu.CompilerParams(needs_layout_passes=False,
                                  use_tc_tiling_on_sc=False)


def _rsqrt16(d):
    i = lax.bitcast_convert_type(d, I32)
    i = jnp.int32(0x5F3759DF) - (i >> 1)
    y = lax.bitcast_convert_type(i, F32)
    for _ in range(3):
        y = y * (1.5 - 0.5 * d * y * y)
    return y


def _sc_deg_body(row_hbm, col_hbm, x_hbm, colp_hbm, dis_hbm, slo_hbm, shi_hbm,
                 row_v, col_v, hist, shared, slab, dis_v, *rest):
    xbuf = rest[0:2]
    xlo = rest[2:4]
    xhi = rest[4:6]
    xsems = rest[6:8]
    lsems = rest[8:10]
    hsems = rest[10:12]
    c = lax.axis_index("c")
    s = lax.axis_index("s")

    zero16 = jnp.zeros((16,), F32)
    ones16 = jnp.ones((16,), F32)
    n16 = jnp.full((16,), N, I32)
    dummy16 = N + lax.iota(I32, 16)

    def zinit(i, carry):
        hist[pl.ds(i * 16, 16)] = zero16
        return carry
    lax.fori_loop(0, NPAD // 16, zinit, 0)

    for h in range(NC):
        pltpu.sync_copy(row_hbm.at[h, s], row_v)
        pltpu.sync_copy(col_hbm.at[h, s], col_v)

        def hbody(j, carry):
            for k in range(C // 16):
                r = row_v[pl.ds(j * C + k * 16, 16)]
                cc = col_v[j, pl.ds(k * 16, 16)]
                m = (r != cc) & (cc < n16)
                plsc.addupdate_scatter(hist, [r], ones16, mask=m)
            return carry
        lax.fori_loop(0, NCHUNK, hbody, 0)

    pltpu.sync_copy(row_hbm.at[c, s], row_v)
    pltpu.sync_copy(col_hbm.at[c, s], col_v)

    def ebody(j, carry):
        for k in range(C // 16):
            r = row_v[pl.ds(j * C + k * 16, 16)]
            cc = col_v[j, pl.ds(k * 16, 16)]
            col_v[j, pl.ds(k * 16, 16)] = jnp.where(r != cc, cc, dummy16)
        return carry
    lax.fori_loop(0, NCHUNK, ebody, 0)
    pltpu.sync_copy(col_v, colp_hbm.at[c, s])

    pltpu.sync_copy(hist, shared.at[s])
    plsc.subcore_barrier()
    for t in range(NS):
        pltpu.sync_copy(shared.at[t, pl.ds(s * RPT, RPT)], slab.at[t])

    def rbody(i, carry):
        a = slab[0, pl.ds(i * 16, 16)]
        for t in range(1, NS):
            a = a + slab[t, pl.ds(i * 16, 16)]
        dis_v[pl.ds(i * 16, 16)] = jnp.where(a > 0.0, _rsqrt16(a), zero16)
        return carry
    lax.fori_loop(0, RPT // 16, rbody, 0)
    pltpu.sync_copy(dis_v.at[pl.ds(0, RPT)], dis_hbm.at[c, pl.ds(s * RPT, RPT)])

    nq = RPT // (2 * C)

    def base(q):
        return s * RPT + 320 * c + q * C

    pltpu.async_copy(x_hbm.at[pl.ds(base(0), C)], xbuf[0], xsems[0])
    for q in range(nq):
        e = q % 2
        if q + 1 < nq:
            pltpu.async_copy(x_hbm.at[pl.ds(base(q + 1), C)],
                             xbuf[1 - e], xsems[1 - e])
        pltpu.make_async_copy(x_hbm.at[pl.ds(base(q), C)], xbuf[e],
                              xsems[e]).wait()
        if q >= 2:
            pltpu.make_async_copy(xlo[e], slo_hbm.at[pl.ds(base(q - 2), C)],
                                  lsems[e]).wait()
            pltpu.make_async_copy(xhi[e], shi_hbm.at[pl.ds(base(q - 2), C)],
                                  hsems[e]).wait()
        l0 = 320 * c + q * C

        def sbody(i, carry):
            dv = dis_v[pl.ds(l0 + i, 16)][0]
            for k in range(D // 16):
                v = xbuf[e][i, pl.ds(k * 16, 16)] * dv
                if k < DH // 16:
                    xlo[e][i, pl.ds(k * 16, 16)] = v
                else:
                    xhi[e][i, pl.ds((k - DH // 16) * 16, 16)] = v
            return carry
        lax.fori_loop(0, C, sbody, 0)
        pltpu.async_copy(xlo[e], slo_hbm.at[pl.ds(base(q), C)], lsems[e])
        pltpu.async_copy(xhi[e], shi_hbm.at[pl.ds(base(q), C)], hsems[e])
    for q in (nq - 2, nq - 1):
        e = q % 2
        pltpu.make_async_copy(xlo[e], slo_hbm.at[pl.ds(base(q), C)],
                              lsems[e]).wait()
        pltpu.make_async_copy(xhi[e], shi_hbm.at[pl.ds(base(q), C)],
                              hsems[e]).wait()


def _make_sc_deg(mesh):
    return pl.kernel(
        _sc_deg_body,
        out_type=(jax.ShapeDtypeStruct((NC, NS, NCHUNK, C), I32),
                  jax.ShapeDtypeStruct((NC, NPAD), F32),
                  jax.ShapeDtypeStruct((NPAD, DH), F32),
                  jax.ShapeDtypeStruct((NPAD, DH), F32)),
        mesh=mesh,
        compiler_params=_SC_PARAMS,
        scratch_types=[
            pltpu.VMEM((EPT,), I32),
            pltpu.VMEM((NCHUNK, C), I32),
            pltpu.VMEM((NPAD,), F32),
            pltpu.VMEM_SHARED((NS, NPAD), F32),
            pltpu.VMEM((NS, RPT), F32),
            pltpu.VMEM((RPT + 16,), F32),
            pltpu.VMEM((C, D), F32),
            pltpu.VMEM((C, D), F32),
            pltpu.VMEM((C, DH), F32),
            pltpu.VMEM((C, DH), F32),
            pltpu.VMEM((C, DH), F32),
            pltpu.VMEM((C, DH), F32),
            pltpu.SemaphoreType.DMA,
            pltpu.SemaphoreType.DMA,
            pltpu.SemaphoreType.DMA,
            pltpu.SemaphoreType.DMA,
            pltpu.SemaphoreType.DMA,
            pltpu.SemaphoreType.DMA,
        ],
    )


def _prop_mainloop(s, table, acc, row_v, colp_v, bufs, gsems, ssems):
    for i in range(RPT // C):
        pltpu.sync_copy(bufs[0], acc.at[pl.ds(s * RPT + i * C, C)])
    plsc.subcore_barrier()

    nround = (NCHUNK + LOOKA + NB) // NB + 1

    def round_(g, carry):
        for b in range(NB):
            k = g * NB + b

            @pl.when((k >= NB) & (k < NCHUNK + NB))
            def _():
                pltpu.make_async_copy(
                    bufs[b], acc.at[colp_v.at[k - NB]], ssems[b]).wait()

            @pl.when(k < NCHUNK)
            def _():
                pltpu.async_copy(
                    table.at[row_v.at[pl.ds(k * C, C)]], bufs[b], gsems[b])

            j = k - LOOKA
            bj = (b - LOOKA) % NB

            @pl.when((j >= 0) & (j < NCHUNK))
            def _():
                pltpu.make_async_copy(
                    table.at[row_v.at[pl.ds(j * C, C)]], bufs[bj],
                    gsems[bj]).wait()
                pltpu.async_copy(bufs[bj], acc.at[colp_v.at[j]],
                                 ssems[bj], add=True)
        return carry
    lax.fori_loop(0, nround, round_, 0)
    plsc.subcore_barrier()


def _zero_seed(buf):
    zero16 = jnp.zeros((16,), F32)

    def zb(i, carry):
        for k in range(DH // 16):
            buf[i, pl.ds(k * 16, 16)] = zero16
        return carry
    lax.fori_loop(0, C, zb, 0)


def _sc_prop1_body(slo_hbm, shi_hbm, row_hbm, colp_hbm, r_hbm,
                   row_v, colp_v, *rest):
    bufs = rest[:NB]
    table, acc = rest[NB], rest[NB + 1]
    gsems = rest[NB + 2:NB + 2 + NB]
    ssems = rest[NB + 2 + NB:]
    c = lax.axis_index("c")
    s = lax.axis_index("s")
    pltpu.sync_copy(row_hbm.at[c, s], row_v)
    pltpu.sync_copy(colp_hbm.at[c, s], colp_v)
    _zero_seed(bufs[0])

    for p, s_hbm in enumerate((slo_hbm, shi_hbm)):
        pltpu.sync_copy(s_hbm.at[pl.ds(s * SPT, SPT)],
                        table.at[pl.ds(s * SPT, SPT)])
        _prop_mainloop(s, table, acc, row_v, colp_v, bufs, gsems, ssems)
        pltpu.sync_copy(acc.at[pl.ds(s * RPT, RPT)],
                        r_hbm.at[c, p, pl.ds(s * RPT, RPT)])
        if p == 0:
            _zero_seed(bufs[0])
            plsc.subcore_barrier()


def _make_sc_prop1(mesh):
    return pl.kernel(
        _sc_prop1_body,
        out_type=jax.ShapeDtypeStruct((NC, 2, NPAD, DH), F32),
        mesh=mesh,
        compiler_params=_SC_PARAMS,
        scratch_types=(
            [pltpu.VMEM((EPT,), I32),
             pltpu.VMEM((NCHUNK, C), I32)]
            + [pltpu.VMEM((C, DH), F32) for _ in range(NB)]
            + [pltpu.VMEM_SHARED((N, DH), F32),
               pltpu.VMEM_SHARED((NPAD, DH), F32)]
            + [pltpu.SemaphoreType.DMA for _ in range(2 * NB)]
        ),
    )


DISW = 656


def _sc_prop2_body(r1_hbm, dis_hbm, row_hbm, colp_hbm, r_hbm,
                   row_v, colp_v, dis_w, *rest):
    bufs = rest[:NB]
    table, acc = rest[NB], rest[NB + 1]
    gsems = rest[NB + 2:NB + 2 + NB]
    ssems = rest[NB + 2 + NB:]
    c = lax.axis_index("c")
    s = lax.axis_index("s")
    pltpu.sync_copy(row_hbm.at[c, s], row_v)
    pltpu.sync_copy(colp_hbm.at[c, s], colp_v)
    pltpu.sync_copy(dis_hbm.at[c, pl.ds(s * 624, DISW)], dis_w)

    nq = SPT // C + 1

    def _cl(q):
        return C if q < nq - 1 else SPT - (nq - 1) * C

    for p in range(2):
        def _ld(q, half, buf, sem):
            return (r1_hbm.at[half, p, pl.ds(s * SPT + q * C, _cl(q))],
                    buf.at[pl.ds(0, _cl(q))], sem)

        def _st(q, buf, sem):
            return (buf.at[pl.ds(0, _cl(q))],
                    table.at[pl.ds(s * SPT + q * C, _cl(q))], sem)

        for q in range(3):
            pltpu.async_copy(*_ld(q, 0, bufs[q], gsems[q]))
        for q in range(2):
            pltpu.async_copy(*_ld(q, 1, bufs[4 + q], gsems[4 + q]))
        for q in range(nq):
            a = q % 4
            bb = 4 + q % 2
            pltpu.make_async_copy(*_ld(q, 0, bufs[a], gsems[a])).wait()
            pltpu.make_async_copy(*_ld(q, 1, bufs[bb], gsems[bb])).wait()

            def tbody(i, carry):
                dv = dis_w[pl.ds(s + q * C + i, 16)][0]
                f = -(dv * dv)
                for k in range(DH // 16):
                    bufs[a][i, pl.ds(k * 16, 16)] = (
                        bufs[a][i, pl.ds(k * 16, 16)]
                        + bufs[bb][i, pl.ds(k * 16, 16)]) * f
                return carry
            lax.fori_loop(0, _cl(q), tbody, 0)
            if q + 2 < nq:
                pltpu.async_copy(*_ld(q + 2, 1, bufs[bb], gsems[bb]))
            pltpu.async_copy(*_st(q, bufs[a], ssems[a]))
            if q + 3 < nq:
                a3 = (q + 3) % 4
                if q >= 1:
                    pltpu.make_async_copy(*_st(q - 1, bufs[a3], ssems[a3])).wait()
                pltpu.async_copy(*_ld(q + 3, 0, bufs[a3], gsems[a3]))
        for q in range(nq - 4, nq):
            a = q % 4
            pltpu.make_async_copy(*_st(q, bufs[a], ssems[a])).wait()

        _zero_seed(bufs[0])
        _prop_mainloop(s, table, acc, row_v, colp_v, bufs, gsems, ssems)
        pltpu.sync_copy(acc.at[pl.ds(s * RPT, RPT)],
                        r_hbm.at[c, p, pl.ds(s * RPT, RPT)])
        if p == 0:
            plsc.subcore_barrier()


def _make_sc_prop2(mesh):
    return pl.kernel(
        _sc_prop2_body,
        out_type=jax.ShapeDtypeStruct((NC, 2, NPAD, DH), F32),
        mesh=mesh,
        compiler_params=_SC_PARAMS,
        scratch_types=(
            [pltpu.VMEM((EPT,), I32),
             pltpu.VMEM((NCHUNK, C), I32),
             pltpu.VMEM((DISW,), F32)]
            + [pltpu.VMEM((C, DH), F32) for _ in range(NB)]
            + [pltpu.VMEM_SHARED((N, DH), F32),
               pltpu.VMEM_SHARED((NPAD, DH), F32)]
            + [pltpu.SemaphoreType.DMA for _ in range(2 * NB)]
        ),
    )


BR = 2000


def _tc_c_body(x_ref, r1_ref, r2_ref, dis_ref, w_ref, b_ref, out_ref):
    x = x_ref[...]
    dis = dis_ref[...]
    tx1 = jnp.concatenate(
        [(r1_ref[0, 0] + r1_ref[1, 0]),
         (r1_ref[0, 1] + r1_ref[1, 1])], axis=1) * (-dis)
    tx2 = jnp.concatenate(
        [(r2_ref[0, 0] + r2_ref[1, 0]),
         (r2_ref[0, 1] + r2_ref[1, 1])], axis=1) * (-2.0 * dis) - x
    out = jnp.dot(x, w_ref[0], preferred_element_type=F32)
    out = out + jnp.dot(tx1, w_ref[1], preferred_element_type=F32)
    out = out + jnp.dot(tx2, w_ref[2], preferred_element_type=F32)
    out_ref[...] = out + b_ref[...]


_tc_c = pl.pallas_call(
    _tc_c_body,
    grid=(N // BR,),
    in_specs=[
        pl.BlockSpec((BR, D), lambda i: (i, 0)),
        pl.BlockSpec((NC, 2, BR, DH), lambda i: (0, 0, i, 0)),
        pl.BlockSpec((NC, 2, BR, DH), lambda i: (0, 0, i, 0)),
        pl.BlockSpec((BR, 1), lambda i: (i, 0)),
        pl.BlockSpec((3, D, D), lambda i: (0, 0, 0)),
        pl.BlockSpec((1, D), lambda i: (0, 0)),
    ],
    out_specs=pl.BlockSpec((BR, D), lambda i: (i, 0)),
    out_shape=jax.ShapeDtypeStruct((N, D), F32),
)


def kernel(x, edge_index, W, b):
    row = edge_index[0].astype(I32)
    col = edge_index[1].astype(I32)
    e = row.shape[0]
    ept_real = e // NTILE
    ppt = EPT - ept_real
    pad_col = (N + jnp.arange(NTILE * ppt, dtype=I32) % (NPAD - N)).reshape(NTILE, ppt)
    row_t = jnp.concatenate(
        [row.reshape(NTILE, ept_real), jnp.zeros((NTILE, ppt), I32)],
        axis=1).reshape(NC, NS, EPT)
    col_t = jnp.concatenate(
        [col.reshape(NTILE, ept_real), pad_col],
        axis=1).reshape(NC, NS, NCHUNK, C)
    x_pad = jnp.concatenate([x, jnp.zeros((NPAD - N, D), F32)])

    mesh = plsc.VectorSubcoreMesh(core_axis_name="c", subcore_axis_name="s")
    colp_t, dis2, slo, shi = _make_sc_deg(mesh)(row_t, col_t, x_pad)
    r1 = _make_sc_prop1(mesh)(slo, shi, row_t, colp_t)
    r2 = _make_sc_prop2(mesh)(r1, dis2, row_t, colp_t)
    out = _tc_c(x, r1, r2, dis2[0, :N].reshape(N, 1), W, b.reshape(1, D))
    return out


# speedup vs baseline: 1.1156x; 1.0175x over previous
import jax
import jax.numpy as jnp
from jax import lax
from jax.experimental import pallas as pl
from jax.experimental.pallas import tpu as pltpu
from jax.experimental.pallas import tpu_sc as plsc

N = 10000
D = 128
DH = D // 2
NC = 2
NS = 16
NTILE = NC * NS
NPAD = 10240
RPT = NPAD // NS
SPT = N // NS
C = 64
EPT = 10240
NCHUNK = EPT // C
EPAD = NTILE * EPT
SW = 64
NB = 6
LOOKA = NB // 2
F32 = jnp.float32
I32 = jnp.int32

_SC_PARAMS = pltpu.CompilerParams(needs_layout_passes=False,
                                  use_tc_tiling_on_sc=False)


def _rsqrt16(d):
    i = lax.bitcast_convert_type(d, I32)
    i = jnp.int32(0x5F3759DF) - (i >> 1)
    y = lax.bitcast_convert_type(i, F32)
    for _ in range(3):
        y = y * (1.5 - 0.5 * d * y * y)
    return y


def _sc_deg_body(row_hbm, col_hbm, x_hbm, colp_hbm, dis_hbm, slo_hbm, shi_hbm,
                 row_v, col_v, hist, shared, slab, dis_v, *rest):
    xbuf = rest[0:2]
    xlo = rest[2:4]
    xhi = rest[4:6]
    xsems = rest[6:8]
    lsems = rest[8:10]
    hsems = rest[10:12]
    c = lax.axis_index("c")
    s = lax.axis_index("s")

    zero16 = jnp.zeros((16,), F32)
    ones16 = jnp.ones((16,), F32)
    n16 = jnp.full((16,), N, I32)
    dummy16 = N + lax.iota(I32, 16)

    def zinit(i, carry):
        hist[pl.ds(i * 16, 16)] = zero16
        return carry
    lax.fori_loop(0, NPAD // 16, zinit, 0)

    for h in range(NC):
        pltpu.sync_copy(row_hbm.at[h, s], row_v)
        pltpu.sync_copy(col_hbm.at[h, s], col_v)

        def hbody(j, carry):
            for k in range(C // 16):
                r = row_v[pl.ds(j * C + k * 16, 16)]
                cc = col_v[j, pl.ds(k * 16, 16)]
                m = (r != cc) & (cc < n16)
                plsc.addupdate_scatter(hist, [r], ones16, mask=m)
            return carry
        lax.fori_loop(0, NCHUNK, hbody, 0)

    pltpu.sync_copy(row_hbm.at[c, s], row_v)
    pltpu.sync_copy(col_hbm.at[c, s], col_v)

    def ebody(j, carry):
        for k in range(C // 16):
            r = row_v[pl.ds(j * C + k * 16, 16)]
            cc = col_v[j, pl.ds(k * 16, 16)]
            col_v[j, pl.ds(k * 16, 16)] = jnp.where(r != cc, cc, dummy16)
        return carry
    lax.fori_loop(0, NCHUNK, ebody, 0)
    pltpu.sync_copy(col_v, colp_hbm.at[c, s])

    pltpu.sync_copy(hist, shared.at[s])
    plsc.subcore_barrier()
    for t in range(NS):
        pltpu.sync_copy(shared.at[t, pl.ds(s * RPT, RPT)], slab.at[t])

    def rbody(i, carry):
        a = slab[0, pl.ds(i * 16, 16)]
        for t in range(1, NS):
            a = a + slab[t, pl.ds(i * 16, 16)]
        dis_v[pl.ds(i * 16, 16)] = jnp.where(a > 0.0, _rsqrt16(a), zero16)
        return carry
    lax.fori_loop(0, RPT // 16, rbody, 0)
    pltpu.sync_copy(dis_v.at[pl.ds(0, RPT)], dis_hbm.at[c, pl.ds(s * RPT, RPT)])

    SW = 64
    nq = RPT // (2 * SW)

    def base(q):
        return jnp.minimum(s * RPT + 320 * c + q * SW, N - SW)

    pltpu.async_copy(x_hbm.at[pl.ds(base(0), SW)], xbuf[0], xsems[0])
    for q in range(nq):
        e = q % 2
        if q + 1 < nq:
            pltpu.async_copy(x_hbm.at[pl.ds(base(q + 1), SW)],
                             xbuf[1 - e], xsems[1 - e])
        pltpu.make_async_copy(x_hbm.at[pl.ds(base(q), SW)], xbuf[e],
                              xsems[e]).wait()
        if q >= 2:
            pltpu.make_async_copy(xlo[e], slo_hbm.at[pl.ds(base(q - 2), SW)],
                                  lsems[e]).wait()
            pltpu.make_async_copy(xhi[e], shi_hbm.at[pl.ds(base(q - 2), SW)],
                                  hsems[e]).wait()
        l0 = base(q) - s * RPT

        def sbody(i, carry):
            dv = dis_v[pl.ds(l0 + i, 16)][0]
            for k in range(D // 16):
                v = xbuf[e][i, pl.ds(k * 16, 16)] * dv
                if k < DH // 16:
                    xlo[e][i, pl.ds(k * 16, 16)] = v
                else:
                    xhi[e][i, pl.ds((k - DH // 16) * 16, 16)] = v
            return carry
        lax.fori_loop(0, SW, sbody, 0)
        pltpu.async_copy(xlo[e], slo_hbm.at[pl.ds(base(q), SW)], lsems[e])
        pltpu.async_copy(xhi[e], shi_hbm.at[pl.ds(base(q), SW)], hsems[e])
    for q in (nq - 2, nq - 1):
        e = q % 2
        pltpu.make_async_copy(xlo[e], slo_hbm.at[pl.ds(base(q), SW)],
                              lsems[e]).wait()
        pltpu.make_async_copy(xhi[e], shi_hbm.at[pl.ds(base(q), SW)],
                              hsems[e]).wait()


def _make_sc_deg(mesh):
    return pl.kernel(
        _sc_deg_body,
        out_type=(jax.ShapeDtypeStruct((NC, NS, NCHUNK, C), I32),
                  jax.ShapeDtypeStruct((NC, NPAD), F32),
                  jax.ShapeDtypeStruct((NPAD, DH), F32),
                  jax.ShapeDtypeStruct((NPAD, DH), F32)),
        mesh=mesh,
        compiler_params=_SC_PARAMS,
        scratch_types=[
            pltpu.VMEM((EPT,), I32),
            pltpu.VMEM((NCHUNK, C), I32),
            pltpu.VMEM((NPAD,), F32),
            pltpu.VMEM_SHARED((NS, NPAD), F32),
            pltpu.VMEM((NS, RPT), F32),
            pltpu.VMEM((RPT + 16,), F32),
            pltpu.VMEM((C, D), F32),
            pltpu.VMEM((C, D), F32),
            pltpu.VMEM((C, DH), F32),
            pltpu.VMEM((C, DH), F32),
            pltpu.VMEM((C, DH), F32),
            pltpu.VMEM((C, DH), F32),
            pltpu.SemaphoreType.DMA,
            pltpu.SemaphoreType.DMA,
            pltpu.SemaphoreType.DMA,
            pltpu.SemaphoreType.DMA,
            pltpu.SemaphoreType.DMA,
            pltpu.SemaphoreType.DMA,
        ],
    )


def _prop_mainloop(s, table, acc, row_v, colp_v, bufs, gsems, ssems):
    for i in range(RPT // C):
        pltpu.sync_copy(bufs[0], acc.at[pl.ds(s * RPT + i * C, C)])
    plsc.subcore_barrier()

    nround = (NCHUNK + LOOKA + NB) // NB + 1

    def round_(g, carry):
        for b in range(NB):
            k = g * NB + b

            @pl.when((k >= NB) & (k < NCHUNK + NB))
            def _():
                pltpu.make_async_copy(
                    bufs[b], acc.at[colp_v.at[k - NB]], ssems[b]).wait()

            @pl.when(k < NCHUNK)
            def _():
                pltpu.async_copy(
                    table.at[row_v.at[pl.ds(k * C, C)]], bufs[b], gsems[b])

            j = k - LOOKA
            bj = (b - LOOKA) % NB

            @pl.when((j >= 0) & (j < NCHUNK))
            def _():
                pltpu.make_async_copy(
                    table.at[row_v.at[pl.ds(j * C, C)]], bufs[bj],
                    gsems[bj]).wait()
                pltpu.async_copy(bufs[bj], acc.at[colp_v.at[j]],
                                 ssems[bj], add=True)
        return carry
    lax.fori_loop(0, nround, round_, 0)
    plsc.subcore_barrier()


def _zero_seed(buf):
    zero16 = jnp.zeros((16,), F32)

    def zb(i, carry):
        for k in range(DH // 16):
            buf[i, pl.ds(k * 16, 16)] = zero16
        return carry
    lax.fori_loop(0, C, zb, 0)


def _sc_prop1_body(slo_hbm, shi_hbm, row_hbm, colp_hbm, r_hbm,
                   row_v, colp_v, *rest):
    bufs = rest[:NB]
    table, acc = rest[NB], rest[NB + 1]
    gsems = rest[NB + 2:NB + 2 + NB]
    ssems = rest[NB + 2 + NB:]
    c = lax.axis_index("c")
    s = lax.axis_index("s")
    pltpu.sync_copy(row_hbm.at[c, s], row_v)
    pltpu.sync_copy(colp_hbm.at[c, s], colp_v)
    _zero_seed(bufs[0])

    for p, s_hbm in enumerate((slo_hbm, shi_hbm)):
        pltpu.sync_copy(s_hbm.at[pl.ds(s * SPT, SPT)],
                        table.at[pl.ds(s * SPT, SPT)])
        _prop_mainloop(s, table, acc, row_v, colp_v, bufs, gsems, ssems)
        pltpu.sync_copy(acc.at[pl.ds(s * RPT, RPT)],
                        r_hbm.at[c, p, pl.ds(s * RPT, RPT)])
        if p == 0:
            _zero_seed(bufs[0])
            plsc.subcore_barrier()


def _make_sc_prop1(mesh):
    return pl.kernel(
        _sc_prop1_body,
        out_type=jax.ShapeDtypeStruct((NC, 2, NPAD, DH), F32),
        mesh=mesh,
        compiler_params=_SC_PARAMS,
        scratch_types=(
            [pltpu.VMEM((EPT,), I32),
             pltpu.VMEM((NCHUNK, C), I32)]
            + [pltpu.VMEM((C, DH), F32) for _ in range(NB)]
            + [pltpu.VMEM_SHARED((N, DH), F32),
               pltpu.VMEM_SHARED((NPAD, DH), F32)]
            + [pltpu.SemaphoreType.DMA for _ in range(2 * NB)]
        ),
    )


DISW = 656


def _sc_prop2_body(r1_hbm, dis_hbm, row_hbm, colp_hbm, r_hbm,
                   row_v, colp_v, dis_w, *rest):
    bufs = rest[:NB]
    table, acc = rest[NB], rest[NB + 1]
    gsems = rest[NB + 2:NB + 2 + NB]
    ssems = rest[NB + 2 + NB:]
    c = lax.axis_index("c")
    s = lax.axis_index("s")
    pltpu.sync_copy(row_hbm.at[c, s], row_v)
    pltpu.sync_copy(colp_hbm.at[c, s], colp_v)
    pltpu.sync_copy(dis_hbm.at[c, pl.ds(s * 624, DISW)], dis_w)

    nq = SPT // C + 1

    def _cl(q):
        return C if q < nq - 1 else SPT - (nq - 1) * C

    for p in range(2):
        def _ld(q, half, buf, sem):
            return (r1_hbm.at[half, p, pl.ds(s * SPT + q * C, _cl(q))],
                    buf.at[pl.ds(0, _cl(q))], sem)

        def _st(q, buf, sem):
            return (buf.at[pl.ds(0, _cl(q))],
                    table.at[pl.ds(s * SPT + q * C, _cl(q))], sem)

        for q in range(3):
            pltpu.async_copy(*_ld(q, 0, bufs[q], gsems[q]))
        for q in range(2):
            pltpu.async_copy(*_ld(q, 1, bufs[4 + q], gsems[4 + q]))
        for q in range(nq):
            a = q % 4
            bb = 4 + q % 2
            pltpu.make_async_copy(*_ld(q, 0, bufs[a], gsems[a])).wait()
            pltpu.make_async_copy(*_ld(q, 1, bufs[bb], gsems[bb])).wait()

            def tbody(i, carry):
                dv = dis_w[pl.ds(s + q * C + i, 16)][0]
                f = -(dv * dv)
                for k in range(DH // 16):
                    bufs[a][i, pl.ds(k * 16, 16)] = (
                        bufs[a][i, pl.ds(k * 16, 16)]
                        + bufs[bb][i, pl.ds(k * 16, 16)]) * f
                return carry
            lax.fori_loop(0, _cl(q), tbody, 0)
            if q + 2 < nq:
                pltpu.async_copy(*_ld(q + 2, 1, bufs[bb], gsems[bb]))
            pltpu.async_copy(*_st(q, bufs[a], ssems[a]))
            if q + 3 < nq:
                a3 = (q + 3) % 4
                if q >= 1:
                    pltpu.make_async_copy(*_st(q - 1, bufs[a3], ssems[a3])).wait()
                pltpu.async_copy(*_ld(q + 3, 0, bufs[a3], gsems[a3]))
        for q in range(nq - 4, nq):
            a = q % 4
            pltpu.make_async_copy(*_st(q, bufs[a], ssems[a])).wait()

        _zero_seed(bufs[0])
        _prop_mainloop(s, table, acc, row_v, colp_v, bufs, gsems, ssems)
        pltpu.sync_copy(acc.at[pl.ds(s * RPT, RPT)],
                        r_hbm.at[c, p, pl.ds(s * RPT, RPT)])
        if p == 0:
            plsc.subcore_barrier()


def _make_sc_prop2(mesh):
    return pl.kernel(
        _sc_prop2_body,
        out_type=jax.ShapeDtypeStruct((NC, 2, NPAD, DH), F32),
        mesh=mesh,
        compiler_params=_SC_PARAMS,
        scratch_types=(
            [pltpu.VMEM((EPT,), I32),
             pltpu.VMEM((NCHUNK, C), I32),
             pltpu.VMEM((DISW,), F32)]
            + [pltpu.VMEM((C, DH), F32) for _ in range(NB)]
            + [pltpu.VMEM_SHARED((N, DH), F32),
               pltpu.VMEM_SHARED((NPAD, DH), F32)]
            + [pltpu.SemaphoreType.DMA for _ in range(2 * NB)]
        ),
    )


BR = 2000


def _tc_c_body(x_ref, r1_ref, r2_ref, dis_ref, w_ref, b_ref, out_ref):
    x = x_ref[...]
    dis = dis_ref[...]
    tx1 = jnp.concatenate(
        [(r1_ref[0, 0] + r1_ref[1, 0]),
         (r1_ref[0, 1] + r1_ref[1, 1])], axis=1) * (-dis)
    tx2 = jnp.concatenate(
        [(r2_ref[0, 0] + r2_ref[1, 0]),
         (r2_ref[0, 1] + r2_ref[1, 1])], axis=1) * (-2.0 * dis) - x
    out = jnp.dot(x, w_ref[0], preferred_element_type=F32)
    out = out + jnp.dot(tx1, w_ref[1], preferred_element_type=F32)
    out = out + jnp.dot(tx2, w_ref[2], preferred_element_type=F32)
    out_ref[...] = out + b_ref[...]


_tc_c = pl.pallas_call(
    _tc_c_body,
    grid=(N // BR,),
    in_specs=[
        pl.BlockSpec((BR, D), lambda i: (i, 0)),
        pl.BlockSpec((NC, 2, BR, DH), lambda i: (0, 0, i, 0)),
        pl.BlockSpec((NC, 2, BR, DH), lambda i: (0, 0, i, 0)),
        pl.BlockSpec((BR, 1), lambda i: (i, 0)),
        pl.BlockSpec((3, D, D), lambda i: (0, 0, 0)),
        pl.BlockSpec((1, D), lambda i: (0, 0)),
    ],
    out_specs=pl.BlockSpec((BR, D), lambda i: (i, 0)),
    out_shape=jax.ShapeDtypeStruct((N, D), F32),
)


def kernel(x, edge_index, W, b):
    row = edge_index[0].astype(I32)
    col = edge_index[1].astype(I32)
    e = row.shape[0]
    ept_real = e // NTILE
    ppt = EPT - ept_real
    pad_col = (N + jnp.arange(NTILE * ppt, dtype=I32) % (NPAD - N)).reshape(NTILE, ppt)
    row_t = jnp.concatenate(
        [row.reshape(NTILE, ept_real), jnp.zeros((NTILE, ppt), I32)],
        axis=1).reshape(NC, NS, EPT)
    col_t = jnp.concatenate(
        [col.reshape(NTILE, ept_real), pad_col],
        axis=1).reshape(NC, NS, NCHUNK, C)

    mesh = plsc.VectorSubcoreMesh(core_axis_name="c", subcore_axis_name="s")
    colp_t, dis2, slo, shi = _make_sc_deg(mesh)(row_t, col_t, x)
    r1 = _make_sc_prop1(mesh)(slo, shi, row_t, colp_t)
    r2 = _make_sc_prop2(mesh)(r1, dis2, row_t, colp_t)
    out = _tc_c(x, r1, r2, dis2[0, :N].reshape(N, 1), W, b.reshape(1, D))
    return out


# speedup vs baseline: 1.1180x; 1.0022x over previous
import jax
import jax.numpy as jnp
from jax import lax
from jax.experimental import pallas as pl
from jax.experimental.pallas import tpu as pltpu
from jax.experimental.pallas import tpu_sc as plsc

N = 10000
D = 128
DH = D // 2
NC = 2
NS = 16
NTILE = NC * NS
NPAD = 10240
RPT = NPAD // NS
SPT = N // NS
C = 64
EPT = 10240
NCHUNK = EPT // C
EPAD = NTILE * EPT
SW = 64
NB = 7
LOOKA = NB // 2
F32 = jnp.float32
I32 = jnp.int32

_SC_PARAMS = pltpu.CompilerParams(needs_layout_passes=False,
                                  use_tc_tiling_on_sc=False)


def _rsqrt16(d):
    i = lax.bitcast_convert_type(d, I32)
    i = jnp.int32(0x5F3759DF) - (i >> 1)
    y = lax.bitcast_convert_type(i, F32)
    for _ in range(3):
        y = y * (1.5 - 0.5 * d * y * y)
    return y


def _sc_deg_body(row_hbm, col_hbm, x_hbm, colp_hbm, dis_hbm, slo_hbm, shi_hbm,
                 row_v, col_v, hist, shared, slab, dis_v, *rest):
    xbuf = rest[0:2]
    xlo = rest[2:4]
    xhi = rest[4:6]
    xsems = rest[6:8]
    lsems = rest[8:10]
    hsems = rest[10:12]
    c = lax.axis_index("c")
    s = lax.axis_index("s")

    zero16 = jnp.zeros((16,), F32)
    ones16 = jnp.ones((16,), F32)
    n16 = jnp.full((16,), N, I32)
    dummy16 = N + lax.iota(I32, 16)

    def zinit(i, carry):
        hist[pl.ds(i * 16, 16)] = zero16
        return carry
    lax.fori_loop(0, NPAD // 16, zinit, 0)

    for h in range(NC):
        pltpu.sync_copy(row_hbm.at[h, s], row_v)
        pltpu.sync_copy(col_hbm.at[h, s], col_v)

        def hbody(j, carry):
            for k in range(C // 16):
                r = row_v[pl.ds(j * C + k * 16, 16)]
                cc = col_v[j, pl.ds(k * 16, 16)]
                m = (r != cc) & (cc < n16)
                plsc.addupdate_scatter(hist, [r], ones16, mask=m)
            return carry
        lax.fori_loop(0, NCHUNK, hbody, 0)

    pltpu.sync_copy(row_hbm.at[c, s], row_v)
    pltpu.sync_copy(col_hbm.at[c, s], col_v)

    def ebody(j, carry):
        for k in range(C // 16):
            r = row_v[pl.ds(j * C + k * 16, 16)]
            cc = col_v[j, pl.ds(k * 16, 16)]
            col_v[j, pl.ds(k * 16, 16)] = jnp.where(r != cc, cc, dummy16)
        return carry
    lax.fori_loop(0, NCHUNK, ebody, 0)
    pltpu.sync_copy(col_v, colp_hbm.at[c, s])

    pltpu.sync_copy(hist, shared.at[s])
    plsc.subcore_barrier()
    for t in range(NS):
        pltpu.sync_copy(shared.at[t, pl.ds(s * RPT, RPT)], slab.at[t])

    def rbody(i, carry):
        a = slab[0, pl.ds(i * 16, 16)]
        for t in range(1, NS):
            a = a + slab[t, pl.ds(i * 16, 16)]
        dis_v[pl.ds(i * 16, 16)] = jnp.where(a > 0.0, _rsqrt16(a), zero16)
        return carry
    lax.fori_loop(0, RPT // 16, rbody, 0)
    pltpu.sync_copy(dis_v.at[pl.ds(0, RPT)], dis_hbm.at[c, pl.ds(s * RPT, RPT)])

    SW = 64
    nq = RPT // (2 * SW)

    def base(q):
        return jnp.minimum(s * RPT + 320 * c + q * SW, N - SW)

    pltpu.async_copy(x_hbm.at[pl.ds(base(0), SW)], xbuf[0], xsems[0])
    for q in range(nq):
        e = q % 2
        if q + 1 < nq:
            pltpu.async_copy(x_hbm.at[pl.ds(base(q + 1), SW)],
                             xbuf[1 - e], xsems[1 - e])
        pltpu.make_async_copy(x_hbm.at[pl.ds(base(q), SW)], xbuf[e],
                              xsems[e]).wait()
        if q >= 2:
            pltpu.make_async_copy(xlo[e], slo_hbm.at[pl.ds(base(q - 2), SW)],
                                  lsems[e]).wait()
            pltpu.make_async_copy(xhi[e], shi_hbm.at[pl.ds(base(q - 2), SW)],
                                  hsems[e]).wait()
        l0 = base(q) - s * RPT

        def sbody(i, carry):
            dv = dis_v[pl.ds(l0 + i, 16)][0]
            for k in range(D // 16):
                v = xbuf[e][i, pl.ds(k * 16, 16)] * dv
                if k < DH // 16:
                    xlo[e][i, pl.ds(k * 16, 16)] = v
                else:
                    xhi[e][i, pl.ds((k - DH // 16) * 16, 16)] = v
            return carry
        lax.fori_loop(0, SW, sbody, 0)
        pltpu.async_copy(xlo[e], slo_hbm.at[pl.ds(base(q), SW)], lsems[e])
        pltpu.async_copy(xhi[e], shi_hbm.at[pl.ds(base(q), SW)], hsems[e])
    for q in (nq - 2, nq - 1):
        e = q % 2
        pltpu.make_async_copy(xlo[e], slo_hbm.at[pl.ds(base(q), SW)],
                              lsems[e]).wait()
        pltpu.make_async_copy(xhi[e], shi_hbm.at[pl.ds(base(q), SW)],
                              hsems[e]).wait()


def _make_sc_deg(mesh):
    return pl.kernel(
        _sc_deg_body,
        out_type=(jax.ShapeDtypeStruct((NC, NS, NCHUNK, C), I32),
                  jax.ShapeDtypeStruct((NC, NPAD), F32),
                  jax.ShapeDtypeStruct((NPAD, DH), F32),
                  jax.ShapeDtypeStruct((NPAD, DH), F32)),
        mesh=mesh,
        compiler_params=_SC_PARAMS,
        scratch_types=[
            pltpu.VMEM((EPT,), I32),
            pltpu.VMEM((NCHUNK, C), I32),
            pltpu.VMEM((NPAD,), F32),
            pltpu.VMEM_SHARED((NS, NPAD), F32),
            pltpu.VMEM((NS, RPT), F32),
            pltpu.VMEM((RPT + 16,), F32),
            pltpu.VMEM((C, D), F32),
            pltpu.VMEM((C, D), F32),
            pltpu.VMEM((C, DH), F32),
            pltpu.VMEM((C, DH), F32),
            pltpu.VMEM((C, DH), F32),
            pltpu.VMEM((C, DH), F32),
            pltpu.SemaphoreType.DMA,
            pltpu.SemaphoreType.DMA,
            pltpu.SemaphoreType.DMA,
            pltpu.SemaphoreType.DMA,
            pltpu.SemaphoreType.DMA,
            pltpu.SemaphoreType.DMA,
        ],
    )


def _prop_mainloop(s, table, acc, row_v, colp_v, bufs, gsems, ssems):
    for i in range(RPT // C):
        pltpu.sync_copy(bufs[0], acc.at[pl.ds(s * RPT + i * C, C)])
    plsc.subcore_barrier()

    nround = (NCHUNK + LOOKA + NB) // NB + 1

    def round_(g, carry):
        for b in range(NB):
            k = g * NB + b

            @pl.when((k >= NB) & (k < NCHUNK + NB))
            def _():
                pltpu.make_async_copy(
                    bufs[b], acc.at[colp_v.at[k - NB]], ssems[b]).wait()

            @pl.when(k < NCHUNK)
            def _():
                pltpu.async_copy(
                    table.at[row_v.at[pl.ds(k * C, C)]], bufs[b], gsems[b])

            j = k - LOOKA
            bj = (b - LOOKA) % NB

            @pl.when((j >= 0) & (j < NCHUNK))
            def _():
                pltpu.make_async_copy(
                    table.at[row_v.at[pl.ds(j * C, C)]], bufs[bj],
                    gsems[bj]).wait()
                pltpu.async_copy(bufs[bj], acc.at[colp_v.at[j]],
                                 ssems[bj], add=True)
        return carry
    lax.fori_loop(0, nround, round_, 0)
    plsc.subcore_barrier()


def _zero_seed(buf):
    zero16 = jnp.zeros((16,), F32)

    def zb(i, carry):
        for k in range(DH // 16):
            buf[i, pl.ds(k * 16, 16)] = zero16
        return carry
    lax.fori_loop(0, C, zb, 0)


def _sc_prop1_body(slo_hbm, shi_hbm, row_hbm, colp_hbm, r_hbm,
                   row_v, colp_v, *rest):
    bufs = rest[:NB]
    table, acc = rest[NB], rest[NB + 1]
    gsems = rest[NB + 2:NB + 2 + NB]
    ssems = rest[NB + 2 + NB:]
    c = lax.axis_index("c")
    s = lax.axis_index("s")
    pltpu.sync_copy(row_hbm.at[c, s], row_v)
    pltpu.sync_copy(colp_hbm.at[c, s], colp_v)
    _zero_seed(bufs[0])

    for p, s_hbm in enumerate((slo_hbm, shi_hbm)):
        pltpu.sync_copy(s_hbm.at[pl.ds(s * SPT, SPT)],
                        table.at[pl.ds(s * SPT, SPT)])
        _prop_mainloop(s, table, acc, row_v, colp_v, bufs, gsems, ssems)
        pltpu.sync_copy(acc.at[pl.ds(s * RPT, RPT)],
                        r_hbm.at[c, p, pl.ds(s * RPT, RPT)])
        if p == 0:
            _zero_seed(bufs[0])
            plsc.subcore_barrier()


def _make_sc_prop1(mesh):
    return pl.kernel(
        _sc_prop1_body,
        out_type=jax.ShapeDtypeStruct((NC, 2, NPAD, DH), F32),
        mesh=mesh,
        compiler_params=_SC_PARAMS,
        scratch_types=(
            [pltpu.VMEM((EPT,), I32),
             pltpu.VMEM((NCHUNK, C), I32)]
            + [pltpu.VMEM((C, DH), F32) for _ in range(NB)]
            + [pltpu.VMEM_SHARED((N, DH), F32),
               pltpu.VMEM_SHARED((NPAD, DH), F32)]
            + [pltpu.SemaphoreType.DMA for _ in range(2 * NB)]
        ),
    )


DISW = 656


def _sc_prop2_body(r1_hbm, dis_hbm, row_hbm, colp_hbm, r_hbm,
                   row_v, colp_v, dis_w, *rest):
    bufs = rest[:NB]
    table, acc = rest[NB], rest[NB + 1]
    gsems = rest[NB + 2:NB + 2 + NB]
    ssems = rest[NB + 2 + NB:]
    c = lax.axis_index("c")
    s = lax.axis_index("s")
    pltpu.sync_copy(row_hbm.at[c, s], row_v)
    pltpu.sync_copy(colp_hbm.at[c, s], colp_v)
    pltpu.sync_copy(dis_hbm.at[c, pl.ds(s * 624, DISW)], dis_w)

    nq = SPT // C + 1

    def _cl(q):
        return C if q < nq - 1 else SPT - (nq - 1) * C

    for p in range(2):
        def _ld(q, half, buf, sem):
            return (r1_hbm.at[half, p, pl.ds(s * SPT + q * C, _cl(q))],
                    buf.at[pl.ds(0, _cl(q))], sem)

        def _st(q, buf, sem):
            return (buf.at[pl.ds(0, _cl(q))],
                    table.at[pl.ds(s * SPT + q * C, _cl(q))], sem)

        for q in range(3):
            pltpu.async_copy(*_ld(q, 0, bufs[q], gsems[q]))
        for q in range(2):
            pltpu.async_copy(*_ld(q, 1, bufs[4 + q], gsems[4 + q]))
        for q in range(nq):
            a = q % 4
            bb = 4 + q % 2
            pltpu.make_async_copy(*_ld(q, 0, bufs[a], gsems[a])).wait()
            pltpu.make_async_copy(*_ld(q, 1, bufs[bb], gsems[bb])).wait()

            def tbody(i, carry):
                dv = dis_w[pl.ds(s + q * C + i, 16)][0]
                f = -(dv * dv)
                for k in range(DH // 16):
                    bufs[a][i, pl.ds(k * 16, 16)] = (
                        bufs[a][i, pl.ds(k * 16, 16)]
                        + bufs[bb][i, pl.ds(k * 16, 16)]) * f
                return carry
            lax.fori_loop(0, _cl(q), tbody, 0)
            if q + 2 < nq:
                pltpu.async_copy(*_ld(q + 2, 1, bufs[bb], gsems[bb]))
            pltpu.async_copy(*_st(q, bufs[a], ssems[a]))
            if q + 3 < nq:
                a3 = (q + 3) % 4
                if q >= 1:
                    pltpu.make_async_copy(*_st(q - 1, bufs[a3], ssems[a3])).wait()
                pltpu.async_copy(*_ld(q + 3, 0, bufs[a3], gsems[a3]))
        for q in range(nq - 4, nq):
            a = q % 4
            pltpu.make_async_copy(*_st(q, bufs[a], ssems[a])).wait()

        _zero_seed(bufs[0])
        _prop_mainloop(s, table, acc, row_v, colp_v, bufs, gsems, ssems)
        pltpu.sync_copy(acc.at[pl.ds(s * RPT, RPT)],
                        r_hbm.at[c, p, pl.ds(s * RPT, RPT)])
        if p == 0:
            plsc.subcore_barrier()


def _make_sc_prop2(mesh):
    return pl.kernel(
        _sc_prop2_body,
        out_type=jax.ShapeDtypeStruct((NC, 2, NPAD, DH), F32),
        mesh=mesh,
        compiler_params=_SC_PARAMS,
        scratch_types=(
            [pltpu.VMEM((EPT,), I32),
             pltpu.VMEM((NCHUNK, C), I32),
             pltpu.VMEM((DISW,), F32)]
            + [pltpu.VMEM((C, DH), F32) for _ in range(NB)]
            + [pltpu.VMEM_SHARED((N, DH), F32),
               pltpu.VMEM_SHARED((NPAD, DH), F32)]
            + [pltpu.SemaphoreType.DMA for _ in range(2 * NB)]
        ),
    )


BR = 2000


def _tc_c_body(x_ref, r1_ref, r2_ref, dis_ref, w_ref, b_ref, out_ref):
    x = x_ref[...]
    dis = dis_ref[...]
    tx1 = jnp.concatenate(
        [(r1_ref[0, 0] + r1_ref[1, 0]),
         (r1_ref[0, 1] + r1_ref[1, 1])], axis=1) * (-dis)
    tx2 = jnp.concatenate(
        [(r2_ref[0, 0] + r2_ref[1, 0]),
         (r2_ref[0, 1] + r2_ref[1, 1])], axis=1) * (-2.0 * dis) - x
    out = jnp.dot(x, w_ref[0], preferred_element_type=F32)
    out = out + jnp.dot(tx1, w_ref[1], preferred_element_type=F32)
    out = out + jnp.dot(tx2, w_ref[2], preferred_element_type=F32)
    out_ref[...] = out + b_ref[...]


_tc_c = pl.pallas_call(
    _tc_c_body,
    grid=(N // BR,),
    in_specs=[
        pl.BlockSpec((BR, D), lambda i: (i, 0)),
        pl.BlockSpec((NC, 2, BR, DH), lambda i: (0, 0, i, 0)),
        pl.BlockSpec((NC, 2, BR, DH), lambda i: (0, 0, i, 0)),
        pl.BlockSpec((BR, 1), lambda i: (i, 0)),
        pl.BlockSpec((3, D, D), lambda i: (0, 0, 0)),
        pl.BlockSpec((1, D), lambda i: (0, 0)),
    ],
    out_specs=pl.BlockSpec((BR, D), lambda i: (i, 0)),
    out_shape=jax.ShapeDtypeStruct((N, D), F32),
)


def kernel(x, edge_index, W, b):
    row = edge_index[0].astype(I32)
    col = edge_index[1].astype(I32)
    e = row.shape[0]
    ept_real = e // NTILE
    ppt = EPT - ept_real
    pad_col = (N + jnp.arange(NTILE * ppt, dtype=I32) % (NPAD - N)).reshape(NTILE, ppt)
    row_t = jnp.concatenate(
        [row.reshape(NTILE, ept_real), jnp.zeros((NTILE, ppt), I32)],
        axis=1).reshape(NC, NS, EPT)
    col_t = jnp.concatenate(
        [col.reshape(NTILE, ept_real), pad_col],
        axis=1).reshape(NC, NS, NCHUNK, C)

    mesh = plsc.VectorSubcoreMesh(core_axis_name="c", subcore_axis_name="s")
    colp_t, dis2, slo, shi = _make_sc_deg(mesh)(row_t, col_t, x)
    r1 = _make_sc_prop1(mesh)(slo, shi, row_t, colp_t)
    r2 = _make_sc_prop2(mesh)(r1, dis2, row_t, colp_t)
    out = _tc_c(x, r1, r2, dis2[0, :N].reshape(N, 1), W, b.reshape(1, D))
    return out
